# baseline (speedup 1.0000x reference)
import functools

import jax
import jax.numpy as jnp
from jax import lax
from jax.experimental import pallas as pl
from jax.experimental.pallas import tpu as pltpu
from jax.experimental.pallas import tpu_sc as plsc

_NW = 32
_ROWS = 16384
_COLS = 64
_RPW = _ROWS // _NW
_LANES = 16


def _sc_partials(x):
    mesh = plsc.VectorSubcoreMesh(core_axis_name="c", subcore_axis_name="s")

    @functools.partial(
        pl.kernel,
        mesh=mesh,
        out_type=jax.ShapeDtypeStruct((_NW, _COLS), jnp.float32),
        scratch_types=[
            pltpu.VMEM((_RPW, _COLS), jnp.float32),
            pltpu.VMEM((_COLS,), jnp.float32),
        ],
    )
    def k(x_hbm, out_hbm, chunk_v, acc_v):
        cid = lax.axis_index("c")
        sid = lax.axis_index("s")
        wid = sid * 2 + cid
        base = wid * _RPW
        pltpu.sync_copy(x_hbm.at[pl.ds(base, _RPW), :], chunk_v)

        def body(r, carry):
            a0, a1, a2, a3 = carry
            a0 = a0 + jnp.maximum(chunk_v[r, pl.ds(0, _LANES)], 0.0)
            a1 = a1 + jnp.maximum(chunk_v[r, pl.ds(16, _LANES)], 0.0)
            a2 = a2 + jnp.maximum(chunk_v[r, pl.ds(32, _LANES)], 0.0)
            a3 = a3 + jnp.maximum(chunk_v[r, pl.ds(48, _LANES)], 0.0)
            return a0, a1, a2, a3

        z = jnp.zeros((_LANES,), jnp.float32)
        a0, a1, a2, a3 = lax.fori_loop(0, _RPW, body, (z, z, z, z))
        acc_v[pl.ds(0, _LANES)] = a0
        acc_v[pl.ds(16, _LANES)] = a1
        acc_v[pl.ds(32, _LANES)] = a2
        acc_v[pl.ds(48, _LANES)] = a3
        pltpu.sync_copy(acc_v, out_hbm.at[wid])

    return k(x)


def _combine(p_ref, o_ref):
    s = jnp.sum(p_ref[...], axis=0, keepdims=True)
    o_ref[...] = jnp.sum(s * s, keepdims=True)


def kernel(x):
    partials = _sc_partials(x)
    out = pl.pallas_call(
        _combine,
        out_shape=jax.ShapeDtypeStruct((1, 1), jnp.float32),
    )(partials)
    return out[0, 0]
